# baseline (speedup 1.0000x reference)
import functools

import jax
import jax.numpy as jnp
from jax import lax
from jax.experimental import pallas as pl
from jax.experimental.pallas import tpu as pltpu
from jax.experimental.pallas import tpu_sc as plsc

N = 10000
NP = 10240
E = 320000
NG = 64
NT = 512
ED = 16
H = 128
D0 = 144
D1 = 128
RB = 512
NB = NP // RB
CH = 128
NTILES = 32
KCH = 80
EP = NTILES * KCH * CH
RPT = NP // 16

_f32 = jnp.float32


def _prep_body(x_ref, emb_ref, out_ref):
    xb = x_ref[...]
    nt = xb[:, 0:1].astype(jnp.int32)
    onehot = (nt == lax.broadcasted_iota(jnp.int32, (RB, NT), 1)).astype(_f32)
    emb_rows = jnp.dot(onehot, emb_ref[...], preferred_element_type=_f32)
    zero_col = jnp.zeros((RB, 1), _f32)
    out_ref[...] = jnp.concatenate([xb[:, 1:], emb_rows, zero_col], axis=-1)


def _mlp_body(h_ref, a0_ref, a1_ref, W1_ref, b1_ref, W2_ref, b2_ref,
              g_ref, be_ref, out_ref):
    z = h_ref[...] + a0_ref[...] + a1_ref[...]
    a = jnp.maximum(jnp.dot(z, W1_ref[...], preferred_element_type=_f32)
                    + b1_ref[...], 0.0)
    o = jnp.dot(a, W2_ref[...], preferred_element_type=_f32) + b2_ref[...]
    mu = jnp.mean(o, axis=-1, keepdims=True)
    var = jnp.mean((o - mu) ** 2, axis=-1, keepdims=True)
    o = (o - mu) * lax.rsqrt(var + 1e-5) * g_ref[...] + be_ref[...]
    out_ref[...] = jnp.maximum(o, 0.0)


def _pool_body(h_ref, bc_ref, gW1_ref, gb1_ref, gW2_ref, gb2_ref,
               cW1_ref, cb1_ref, cW2_ref, cb2_ref, out_ref,
               gate_s, gmax_s, den_s, pool_s):
    p = pl.program_id(0)
    i = pl.program_id(1)
    hb = h_ref[...]
    bc = bc_ref[...]
    seg_ids = lax.broadcasted_iota(jnp.int32, (RB, NG), 1).astype(_f32)
    seg = (bc == seg_ids).astype(_f32)

    @pl.when(p == 0)
    def _phase_gate_max():
        g = jnp.dot(jnp.maximum(jnp.dot(hb, gW1_ref[...],
                                        preferred_element_type=_f32)
                                + gb1_ref[...], 0.0),
                    gW2_ref[...], preferred_element_type=_f32) + gb2_ref[...]
        gate_s[pl.ds(i * RB, RB), :] = g
        m = jnp.max(jnp.where(seg > 0.0, g, -1e30), axis=0, keepdims=True)
        prev = jnp.where(i == 0, jnp.full((1, NG), -1e30, _f32), gmax_s[...])
        gmax_s[...] = jnp.maximum(prev, m)

    @pl.when(p == 1)
    def _phase_denom():
        g = gate_s[pl.ds(i * RB, RB), :]
        gmaxn = jnp.sum(seg * gmax_s[...], axis=1, keepdims=True)
        alpha = jnp.exp(g - gmaxn) * seg.max(axis=1, keepdims=True)
        prev = jnp.where(i == 0, jnp.zeros((1, NG), _f32), den_s[...])
        den_s[...] = prev + jnp.sum(seg * alpha, axis=0, keepdims=True)

    @pl.when(p == 2)
    def _phase_weighted_sum():
        g = gate_s[pl.ds(i * RB, RB), :]
        gmaxn = jnp.sum(seg * gmax_s[...], axis=1, keepdims=True)
        alpha = jnp.exp(g - gmaxn)
        denn = jnp.sum(seg * den_s[...], axis=1, keepdims=True)
        w = alpha / (denn + 1e-16)
        contrib = lax.dot_general(seg, w * hb, (((0,), (0,)), ((), ())),
                                  preferred_element_type=_f32)
        prev = jnp.where(i == 0, jnp.zeros((NG, H), _f32), pool_s[...])
        pool_s[...] = prev + contrib

    @pl.when((p == 2) & (i == NB - 1))
    def _classify():
        pooled = pool_s[...]
        zc = jnp.maximum(jnp.dot(pooled, cW1_ref[...],
                                 preferred_element_type=_f32) + cb1_ref[...], 0.0)
        out_ref[...] = jnp.dot(zc, cW2_ref[...],
                               preferred_element_type=_f32) + cb2_ref[...]


def _make_edge_agg(D):
    mesh = plsc.VectorSubcoreMesh(core_axis_name="c", subcore_axis_name="s")

    @functools.partial(
        pl.kernel,
        out_type=jax.ShapeDtypeStruct((2 * NP, D), _f32),
        mesh=mesh,
        compiler_params=pltpu.CompilerParams(use_tc_tiling_on_sc=False),
        scratch_types=[
            pltpu.VMEM((CH,), jnp.int32),
            pltpu.VMEM((CH,), jnp.int32),
            pltpu.VMEM((CH, D), _f32),
            pltpu.VMEM_SHARED((NP, D), _f32),
            pltpu.SemaphoreType.DMA,
        ],
    )
    def edge_agg(h_hbm, src_hbm, dst_hbm, zeros_hbm, out_hbm,
                 src_v, dst_v, rows_v, agg_sh, sem):
        c = lax.axis_index("c")
        s = lax.axis_index("s")
        wid = s * 2 + c
        rbase = s * RPT

        pltpu.sync_copy(zeros_hbm, rows_v)
        def zero_body(m, carry):
            pltpu.sync_copy(rows_v, agg_sh.at[pl.ds(rbase + m * CH, CH)])
            return carry
        lax.fori_loop(0, RPT // CH, zero_body, 0)
        plsc.subcore_barrier()

        ebase = wid * (KCH * CH)
        def edge_body(j, carry):
            off = ebase + j * CH
            pltpu.sync_copy(src_hbm.at[pl.ds(off, CH)], src_v)
            pltpu.async_copy(h_hbm.at[src_v], rows_v, sem).wait()
            pltpu.sync_copy(dst_hbm.at[pl.ds(off, CH)], dst_v)
            pltpu.sync_copy(rows_v, agg_sh.at[dst_v], add=True)
            return carry
        lax.fori_loop(0, KCH, edge_body, 0)
        plsc.subcore_barrier()

        pltpu.sync_copy(agg_sh.at[pl.ds(rbase, RPT)],
                        out_hbm.at[pl.ds(c * NP + rbase, RPT)])

    return edge_agg


def _row_spec(D):
    return pl.BlockSpec((RB, D), lambda i: (i, 0))


def _full(shape):
    return pl.BlockSpec(shape, lambda i: (0, 0))


def kernel(x, edge_index, batch, emb,
           conv0_W1, conv0_b1, conv0_W2, conv0_b2,
           conv1_W1, conv1_b1, conv1_W2, conv1_b2,
           ln0_g, ln0_b, ln1_g, ln1_b,
           gate_W1, gate_b1, gate_W2, gate_b2,
           cls_W1, cls_b1, cls_W2, cls_b2):
    x_p = jnp.pad(x, ((0, NP - N), (0, 0)))
    emb_p = jnp.pad(emb, ((0, NT - emb.shape[0]), (0, 0)))
    src_p = jnp.concatenate([edge_index[0], jnp.zeros((EP - E,), jnp.int32)])
    dead = N + (jnp.arange(EP - E, dtype=jnp.int32) % (NP - N))
    dst_p = jnp.concatenate([edge_index[1], dead])
    bc = jnp.pad(batch.astype(_f32), (0, NP - N),
                 constant_values=float(NG))[:, None]
    W1_0 = jnp.pad(conv0_W1, ((0, D0 - conv0_W1.shape[0]), (0, 0)))
    z0 = jnp.zeros((CH, D0), _f32)
    z1 = jnp.zeros((CH, D1), _f32)
    row2 = lambda v: v[None, :]

    h0 = pl.pallas_call(
        _prep_body,
        grid=(NB,),
        in_specs=[_row_spec(128), _full((NT, ED))],
        out_specs=_row_spec(D0),
        out_shape=jax.ShapeDtypeStruct((NP, D0), _f32),
    )(x_p, emb_p)

    def gin_layer(h, D, W1, b1, W2, b2, g, be):
        parts = _make_edge_agg(D)(h, src_p, dst_p, z0 if D == D0 else z1)
        return pl.pallas_call(
            _mlp_body,
            grid=(NB,),
            in_specs=[
                _row_spec(D),
                pl.BlockSpec((RB, D), lambda i: (i, 0)),
                pl.BlockSpec((RB, D), lambda i: (i + NB, 0)),
                _full((D, H)), _full((1, H)), _full((H, H)), _full((1, H)),
                _full((1, H)), _full((1, H)),
            ],
            out_specs=_row_spec(H),
            out_shape=jax.ShapeDtypeStruct((NP, H), _f32),
        )(h, parts, parts, W1, row2(b1), W2, row2(b2), row2(g), row2(be))

    h1 = gin_layer(h0, D0, W1_0, conv0_b1, conv0_W2, conv0_b2, ln0_g, ln0_b)
    h2 = gin_layer(h1, D1, conv1_W1, conv1_b1, conv1_W2, conv1_b2, ln1_g, ln1_b)

    out = pl.pallas_call(
        _pool_body,
        grid=(3, NB),
        in_specs=[
            pl.BlockSpec((RB, H), lambda p, i: (i, 0)),
            pl.BlockSpec((RB, 1), lambda p, i: (i, 0)),
            pl.BlockSpec((H, H), lambda p, i: (0, 0)),
            pl.BlockSpec((1, H), lambda p, i: (0, 0)),
            pl.BlockSpec((H, 1), lambda p, i: (0, 0)),
            pl.BlockSpec((1, 1), lambda p, i: (0, 0)),
            pl.BlockSpec((H, H), lambda p, i: (0, 0)),
            pl.BlockSpec((1, H), lambda p, i: (0, 0)),
            pl.BlockSpec((H, 2), lambda p, i: (0, 0)),
            pl.BlockSpec((1, 2), lambda p, i: (0, 0)),
        ],
        out_specs=pl.BlockSpec((NG, 2), lambda p, i: (0, 0)),
        out_shape=jax.ShapeDtypeStruct((NG, 2), _f32),
        scratch_shapes=[
            pltpu.VMEM((NP, 1), _f32),
            pltpu.VMEM((1, NG), _f32),
            pltpu.VMEM((1, NG), _f32),
            pltpu.VMEM((NG, H), _f32),
        ],
    )(h2, bc, gate_W1, row2(gate_b1), gate_W2, row2(gate_b2),
      cls_W1, row2(cls_b1), cls_W2, row2(cls_b2))
    return out


# speedup vs baseline: 1.4464x; 1.4464x over previous
import functools

import jax
import jax.numpy as jnp
from jax import lax
from jax.experimental import pallas as pl
from jax.experimental.pallas import tpu as pltpu
from jax.experimental.pallas import tpu_sc as plsc

N = 10000
NP = 10240
E = 320000
NG = 64
NT = 512
ED = 16
H = 128
D0 = 144
D1 = 128
RB = 512
NB = NP // RB
CH = 128
NTILES = 32
KCH = 79
EP = NTILES * KCH * CH
RPT = NP // 16

_f32 = jnp.float32


def _prep_body(x_ref, emb_ref, out_ref):
    xb = x_ref[...]
    nt = xb[:, 0:1].astype(jnp.int32)
    onehot = (nt == lax.broadcasted_iota(jnp.int32, (RB, NT), 1)).astype(_f32)
    emb_rows = jnp.dot(onehot, emb_ref[...], preferred_element_type=_f32)
    zero_col = jnp.zeros((RB, 1), _f32)
    out_ref[...] = jnp.concatenate([xb[:, 1:], emb_rows, zero_col], axis=-1)


def _mlp_body(h_ref, a0_ref, a1_ref, W1_ref, b1_ref, W2_ref, b2_ref,
              g_ref, be_ref, out_ref):
    z = h_ref[...] + a0_ref[...] + a1_ref[...]
    a = jnp.maximum(jnp.dot(z, W1_ref[...], preferred_element_type=_f32)
                    + b1_ref[...], 0.0)
    o = jnp.dot(a, W2_ref[...], preferred_element_type=_f32) + b2_ref[...]
    mu = jnp.mean(o, axis=-1, keepdims=True)
    var = jnp.mean((o - mu) ** 2, axis=-1, keepdims=True)
    o = (o - mu) * lax.rsqrt(var + 1e-5) * g_ref[...] + be_ref[...]
    out_ref[...] = jnp.maximum(o, 0.0)


def _pool_body(h_ref, bc_ref, gW1_ref, gb1_ref, gW2_ref, gb2_ref,
               cW1_ref, cb1_ref, cW2_ref, cb2_ref, out_ref,
               gate_s, gmax_s, den_s, pool_s):
    p = pl.program_id(0)
    i = pl.program_id(1)
    hb = h_ref[...]
    bc = bc_ref[...]
    seg_ids = lax.broadcasted_iota(jnp.int32, (RB, NG), 1).astype(_f32)
    seg = (bc == seg_ids).astype(_f32)

    @pl.when(p == 0)
    def _phase_gate_max():
        g = jnp.dot(jnp.maximum(jnp.dot(hb, gW1_ref[...],
                                        preferred_element_type=_f32)
                                + gb1_ref[...], 0.0),
                    gW2_ref[...], preferred_element_type=_f32) + gb2_ref[...]
        gate_s[pl.ds(i * RB, RB), :] = g
        m = jnp.max(jnp.where(seg > 0.0, g, -1e30), axis=0, keepdims=True)
        prev = jnp.where(i == 0, jnp.full((1, NG), -1e30, _f32), gmax_s[...])
        gmax_s[...] = jnp.maximum(prev, m)

    @pl.when(p == 1)
    def _phase_denom():
        g = gate_s[pl.ds(i * RB, RB), :]
        gmaxn = jnp.sum(seg * gmax_s[...], axis=1, keepdims=True)
        alpha = jnp.exp(g - gmaxn) * seg.max(axis=1, keepdims=True)
        prev = jnp.where(i == 0, jnp.zeros((1, NG), _f32), den_s[...])
        den_s[...] = prev + jnp.sum(seg * alpha, axis=0, keepdims=True)

    @pl.when(p == 2)
    def _phase_weighted_sum():
        g = gate_s[pl.ds(i * RB, RB), :]
        gmaxn = jnp.sum(seg * gmax_s[...], axis=1, keepdims=True)
        alpha = jnp.exp(g - gmaxn)
        denn = jnp.sum(seg * den_s[...], axis=1, keepdims=True)
        w = alpha / (denn + 1e-16)
        contrib = lax.dot_general(seg, w * hb, (((0,), (0,)), ((), ())),
                                  preferred_element_type=_f32)
        prev = jnp.where(i == 0, jnp.zeros((NG, H), _f32), pool_s[...])
        pool_s[...] = prev + contrib

    @pl.when((p == 2) & (i == NB - 1))
    def _classify():
        pooled = pool_s[...]
        zc = jnp.maximum(jnp.dot(pooled, cW1_ref[...],
                                 preferred_element_type=_f32) + cb1_ref[...], 0.0)
        out_ref[...] = jnp.dot(zc, cW2_ref[...],
                               preferred_element_type=_f32) + cb2_ref[...]


def _make_edge_agg(D):
    mesh = plsc.VectorSubcoreMesh(core_axis_name="c", subcore_axis_name="s")

    @functools.partial(
        pl.kernel,
        out_type=jax.ShapeDtypeStruct((2 * NP, D), _f32),
        mesh=mesh,
        compiler_params=pltpu.CompilerParams(use_tc_tiling_on_sc=False),
        scratch_types=[
            pltpu.VMEM((CH,), jnp.int32),
            pltpu.VMEM((CH,), jnp.int32),
            pltpu.VMEM((CH, D), _f32),
            pltpu.VMEM_SHARED((NP, D), _f32),
            pltpu.SemaphoreType.DMA,
        ],
    )
    def edge_agg(h_hbm, src_hbm, dst_hbm, zeros_hbm, out_hbm,
                 src_v, dst_v, rows_v, agg_sh, sem):
        c = lax.axis_index("c")
        s = lax.axis_index("s")
        wid = s * 2 + c
        rbase = s * RPT

        pltpu.sync_copy(zeros_hbm, rows_v)
        def zero_body(m, carry):
            pltpu.sync_copy(rows_v, agg_sh.at[pl.ds(rbase + m * CH, CH)])
            return carry
        lax.fori_loop(0, RPT // CH, zero_body, 0)
        plsc.subcore_barrier()

        ebase = wid * (KCH * CH)
        def edge_body(j, carry):
            off = ebase + j * CH
            pltpu.sync_copy(src_hbm.at[pl.ds(off, CH)], src_v)
            pltpu.async_copy(h_hbm.at[src_v], rows_v, sem).wait()
            pltpu.sync_copy(dst_hbm.at[pl.ds(off, CH)], dst_v)
            pltpu.sync_copy(rows_v, agg_sh.at[dst_v], add=True)
            return carry
        lax.fori_loop(0, KCH, edge_body, 0)
        plsc.subcore_barrier()

        pltpu.sync_copy(agg_sh.at[pl.ds(rbase, RPT)],
                        out_hbm.at[pl.ds(c * NP + rbase, RPT)])

    return edge_agg


def _row_spec(D):
    return pl.BlockSpec((RB, D), lambda i: (i, 0))


def _full(shape):
    return pl.BlockSpec(shape, lambda i: (0, 0))


def kernel(x, edge_index, batch, emb,
           conv0_W1, conv0_b1, conv0_W2, conv0_b2,
           conv1_W1, conv1_b1, conv1_W2, conv1_b2,
           ln0_g, ln0_b, ln1_g, ln1_b,
           gate_W1, gate_b1, gate_W2, gate_b2,
           cls_W1, cls_b1, cls_W2, cls_b2):
    x_p = jnp.pad(x, ((0, NP - N), (0, 0)))
    emb_p = jnp.pad(emb, ((0, NT - emb.shape[0]), (0, 0)))
    pt = KCH * CH - E // NTILES
    src_p = jnp.pad(edge_index[0].reshape(NTILES, E // NTILES),
                    ((0, 0), (0, pt))).reshape(-1)
    dst_p = jnp.pad(edge_index[1].reshape(NTILES, E // NTILES),
                    ((0, 0), (0, pt)), constant_values=N).reshape(-1)
    bc = jnp.pad(batch.astype(_f32), (0, NP - N),
                 constant_values=float(NG))[:, None]
    W1_0 = jnp.pad(conv0_W1, ((0, D0 - conv0_W1.shape[0]), (0, 0)))
    z0 = jnp.zeros((CH, D0), _f32)
    z1 = jnp.zeros((CH, D1), _f32)
    row2 = lambda v: v[None, :]

    h0 = pl.pallas_call(
        _prep_body,
        grid=(NB,),
        in_specs=[_row_spec(128), _full((NT, ED))],
        out_specs=_row_spec(D0),
        out_shape=jax.ShapeDtypeStruct((NP, D0), _f32),
    )(x_p, emb_p)

    def gin_layer(h, D, W1, b1, W2, b2, g, be):
        parts = _make_edge_agg(D)(h, src_p, dst_p, z0 if D == D0 else z1)
        return pl.pallas_call(
            _mlp_body,
            grid=(NB,),
            in_specs=[
                _row_spec(D),
                pl.BlockSpec((RB, D), lambda i: (i, 0)),
                pl.BlockSpec((RB, D), lambda i: (i + NB, 0)),
                _full((D, H)), _full((1, H)), _full((H, H)), _full((1, H)),
                _full((1, H)), _full((1, H)),
            ],
            out_specs=_row_spec(H),
            out_shape=jax.ShapeDtypeStruct((NP, H), _f32),
        )(h, parts, parts, W1, row2(b1), W2, row2(b2), row2(g), row2(be))

    h1 = gin_layer(h0, D0, W1_0, conv0_b1, conv0_W2, conv0_b2, ln0_g, ln0_b)
    h2 = gin_layer(h1, D1, conv1_W1, conv1_b1, conv1_W2, conv1_b2, ln1_g, ln1_b)

    out = pl.pallas_call(
        _pool_body,
        grid=(3, NB),
        in_specs=[
            pl.BlockSpec((RB, H), lambda p, i: (i, 0)),
            pl.BlockSpec((RB, 1), lambda p, i: (i, 0)),
            pl.BlockSpec((H, H), lambda p, i: (0, 0)),
            pl.BlockSpec((1, H), lambda p, i: (0, 0)),
            pl.BlockSpec((H, 1), lambda p, i: (0, 0)),
            pl.BlockSpec((1, 1), lambda p, i: (0, 0)),
            pl.BlockSpec((H, H), lambda p, i: (0, 0)),
            pl.BlockSpec((1, H), lambda p, i: (0, 0)),
            pl.BlockSpec((H, 2), lambda p, i: (0, 0)),
            pl.BlockSpec((1, 2), lambda p, i: (0, 0)),
        ],
        out_specs=pl.BlockSpec((NG, 2), lambda p, i: (0, 0)),
        out_shape=jax.ShapeDtypeStruct((NG, 2), _f32),
        scratch_shapes=[
            pltpu.VMEM((NP, 1), _f32),
            pltpu.VMEM((1, NG), _f32),
            pltpu.VMEM((1, NG), _f32),
            pltpu.VMEM((NG, H), _f32),
        ],
    )(h2, bc, gate_W1, row2(gate_b1), gate_W2, row2(gate_b2),
      cls_W1, row2(cls_b1), cls_W2, row2(cls_b2))
    return out


# speedup vs baseline: 1.8291x; 1.2645x over previous
import functools

import jax
import jax.numpy as jnp
from jax import lax
from jax.experimental import pallas as pl
from jax.experimental.pallas import tpu as pltpu
from jax.experimental.pallas import tpu_sc as plsc

N = 10000
NP = 10240
E = 320000
NG = 64
NT = 512
ED = 16
H = 128
D0 = 144
D1 = 128
RB = 512
NB = NP // RB
CH = 128
NTILES = 32
KCH = 79
EP = NTILES * KCH * CH
RPT = NP // 16

_f32 = jnp.float32


def _prep_body(x_ref, emb_ref, out_ref):
    xb = x_ref[...]
    nt = xb[:, 0:1].astype(jnp.int32)
    onehot = (nt == lax.broadcasted_iota(jnp.int32, (RB, NT), 1)).astype(_f32)
    emb_rows = jnp.dot(onehot, emb_ref[...], preferred_element_type=_f32)
    zero_col = jnp.zeros((RB, 1), _f32)
    out_ref[...] = jnp.concatenate([xb[:, 1:], emb_rows, zero_col], axis=-1)


def _mlp_body(h_ref, a0_ref, a1_ref, W1_ref, b1_ref, W2_ref, b2_ref,
              g_ref, be_ref, out_ref):
    z = h_ref[...] + a0_ref[...] + a1_ref[...]
    a = jnp.maximum(jnp.dot(z, W1_ref[...], preferred_element_type=_f32)
                    + b1_ref[...], 0.0)
    o = jnp.dot(a, W2_ref[...], preferred_element_type=_f32) + b2_ref[...]
    mu = jnp.mean(o, axis=-1, keepdims=True)
    var = jnp.mean((o - mu) ** 2, axis=-1, keepdims=True)
    o = (o - mu) * lax.rsqrt(var + 1e-5) * g_ref[...] + be_ref[...]
    out_ref[...] = jnp.maximum(o, 0.0)


def _pool_body(h_ref, bc_ref, gW1_ref, gb1_ref, gW2_ref, gb2_ref,
               cW1_ref, cb1_ref, cW2_ref, cb2_ref, out_ref,
               gate_s, gmax_s, den_s, pool_s):
    p = pl.program_id(0)
    i = pl.program_id(1)
    hb = h_ref[...]
    bc = bc_ref[...]
    seg_ids = lax.broadcasted_iota(jnp.int32, (RB, NG), 1).astype(_f32)
    seg = (bc == seg_ids).astype(_f32)

    @pl.when(p == 0)
    def _phase_gate_max():
        g = jnp.dot(jnp.maximum(jnp.dot(hb, gW1_ref[...],
                                        preferred_element_type=_f32)
                                + gb1_ref[...], 0.0),
                    gW2_ref[...], preferred_element_type=_f32) + gb2_ref[...]
        gate_s[pl.ds(i * RB, RB), :] = g
        m = jnp.max(jnp.where(seg > 0.0, g, -1e30), axis=0, keepdims=True)
        prev = jnp.where(i == 0, jnp.full((1, NG), -1e30, _f32), gmax_s[...])
        gmax_s[...] = jnp.maximum(prev, m)

    @pl.when(p == 1)
    def _phase_denom():
        g = gate_s[pl.ds(i * RB, RB), :]
        gmaxn = jnp.sum(seg * gmax_s[...], axis=1, keepdims=True)
        alpha = jnp.exp(g - gmaxn) * seg.max(axis=1, keepdims=True)
        prev = jnp.where(i == 0, jnp.zeros((1, NG), _f32), den_s[...])
        den_s[...] = prev + jnp.sum(seg * alpha, axis=0, keepdims=True)

    @pl.when(p == 2)
    def _phase_weighted_sum():
        g = gate_s[pl.ds(i * RB, RB), :]
        gmaxn = jnp.sum(seg * gmax_s[...], axis=1, keepdims=True)
        alpha = jnp.exp(g - gmaxn)
        denn = jnp.sum(seg * den_s[...], axis=1, keepdims=True)
        w = alpha / (denn + 1e-16)
        contrib = lax.dot_general(seg, w * hb, (((0,), (0,)), ((), ())),
                                  preferred_element_type=_f32)
        prev = jnp.where(i == 0, jnp.zeros((NG, H), _f32), pool_s[...])
        pool_s[...] = prev + contrib

    @pl.when((p == 2) & (i == NB - 1))
    def _classify():
        pooled = pool_s[...]
        zc = jnp.maximum(jnp.dot(pooled, cW1_ref[...],
                                 preferred_element_type=_f32) + cb1_ref[...], 0.0)
        out_ref[...] = jnp.dot(zc, cW2_ref[...],
                               preferred_element_type=_f32) + cb2_ref[...]


def _make_edge_agg(D):
    mesh = plsc.VectorSubcoreMesh(core_axis_name="c", subcore_axis_name="s")

    @functools.partial(
        pl.kernel,
        out_type=jax.ShapeDtypeStruct((2 * NP, D), _f32),
        mesh=mesh,
        compiler_params=pltpu.CompilerParams(use_tc_tiling_on_sc=False),
        scratch_types=[
            pltpu.VMEM((CH,), jnp.int32),
            pltpu.VMEM((CH,), jnp.int32),
            pltpu.VMEM((CH,), jnp.int32),
            pltpu.VMEM((CH,), jnp.int32),
            pltpu.VMEM((CH, D), _f32),
            pltpu.VMEM((CH, D), _f32),
            pltpu.VMEM_SHARED((NP, D), _f32),
            pltpu.SemaphoreType.DMA,
            pltpu.SemaphoreType.DMA,
        ],
    )
    def edge_agg(h_hbm, src_hbm, dst_hbm, zeros_hbm, out_hbm,
                 src_v0, src_v1, dst_v0, dst_v1, rows0, rows1, agg_sh,
                 sem0, sem1):
        c = lax.axis_index("c")
        s = lax.axis_index("s")
        wid = s * 2 + c
        rbase = s * RPT
        ebase = wid * (KCH * CH)

        pltpu.sync_copy(zeros_hbm, rows0)
        def zero_body(m, carry):
            pltpu.sync_copy(rows0, agg_sh.at[pl.ds(rbase + m * CH, CH)])
            return carry
        lax.fori_loop(0, RPT // CH, zero_body, 0)

        pltpu.sync_copy(src_hbm.at[pl.ds(ebase, CH)], src_v0)
        pltpu.async_copy(h_hbm.at[src_v0], rows0, sem0)
        plsc.subcore_barrier()

        def edge_body(i, carry):
            j1 = 2 * i + 1
            j2 = j1 + 1
            pltpu.sync_copy(src_hbm.at[pl.ds(ebase + j1 * CH, CH)], src_v1)
            pltpu.async_copy(h_hbm.at[src_v1], rows1, sem1)
            pltpu.sync_copy(dst_hbm.at[pl.ds(ebase + (j1 - 1) * CH, CH)], dst_v0)
            pltpu.make_async_copy(h_hbm.at[src_v0], rows0, sem0).wait()
            pltpu.sync_copy(rows0, agg_sh.at[dst_v0], add=True)
            @pl.when(j2 < KCH)
            def _():
                pltpu.sync_copy(src_hbm.at[pl.ds(ebase + j2 * CH, CH)], src_v0)
                pltpu.async_copy(h_hbm.at[src_v0], rows0, sem0)
            pltpu.sync_copy(dst_hbm.at[pl.ds(ebase + j1 * CH, CH)], dst_v1)
            pltpu.make_async_copy(h_hbm.at[src_v1], rows1, sem1).wait()
            pltpu.sync_copy(rows1, agg_sh.at[dst_v1], add=True)
            return carry
        lax.fori_loop(0, KCH // 2, edge_body, 0)

        jt = KCH - 1
        pltpu.sync_copy(dst_hbm.at[pl.ds(ebase + jt * CH, CH)], dst_v0)
        pltpu.make_async_copy(h_hbm.at[src_v0], rows0, sem0).wait()
        pltpu.sync_copy(rows0, agg_sh.at[dst_v0], add=True)
        plsc.subcore_barrier()

        pltpu.sync_copy(agg_sh.at[pl.ds(rbase, RPT)],
                        out_hbm.at[pl.ds(c * NP + rbase, RPT)])

    return edge_agg


def _row_spec(D):
    return pl.BlockSpec((RB, D), lambda i: (i, 0))


def _full(shape):
    return pl.BlockSpec(shape, lambda i: (0, 0))


def kernel(x, edge_index, batch, emb,
           conv0_W1, conv0_b1, conv0_W2, conv0_b2,
           conv1_W1, conv1_b1, conv1_W2, conv1_b2,
           ln0_g, ln0_b, ln1_g, ln1_b,
           gate_W1, gate_b1, gate_W2, gate_b2,
           cls_W1, cls_b1, cls_W2, cls_b2):
    x_p = jnp.pad(x, ((0, NP - N), (0, 0)))
    emb_p = jnp.pad(emb, ((0, NT - emb.shape[0]), (0, 0)))
    pt = KCH * CH - E // NTILES
    src_p = jnp.pad(edge_index[0].reshape(NTILES, E // NTILES),
                    ((0, 0), (0, pt))).reshape(-1)
    dst_p = jnp.pad(edge_index[1].reshape(NTILES, E // NTILES),
                    ((0, 0), (0, pt)), constant_values=N).reshape(-1)
    bc = jnp.pad(batch.astype(_f32), (0, NP - N),
                 constant_values=float(NG))[:, None]
    W1_0 = jnp.pad(conv0_W1, ((0, D0 - conv0_W1.shape[0]), (0, 0)))
    z0 = jnp.zeros((CH, D0), _f32)
    z1 = jnp.zeros((CH, D1), _f32)
    row2 = lambda v: v[None, :]

    h0 = pl.pallas_call(
        _prep_body,
        grid=(NB,),
        in_specs=[_row_spec(128), _full((NT, ED))],
        out_specs=_row_spec(D0),
        out_shape=jax.ShapeDtypeStruct((NP, D0), _f32),
    )(x_p, emb_p)

    def gin_layer(h, D, W1, b1, W2, b2, g, be):
        parts = _make_edge_agg(D)(h, src_p, dst_p, z0 if D == D0 else z1)
        return pl.pallas_call(
            _mlp_body,
            grid=(NB,),
            in_specs=[
                _row_spec(D),
                pl.BlockSpec((RB, D), lambda i: (i, 0)),
                pl.BlockSpec((RB, D), lambda i: (i + NB, 0)),
                _full((D, H)), _full((1, H)), _full((H, H)), _full((1, H)),
                _full((1, H)), _full((1, H)),
            ],
            out_specs=_row_spec(H),
            out_shape=jax.ShapeDtypeStruct((NP, H), _f32),
        )(h, parts, parts, W1, row2(b1), W2, row2(b2), row2(g), row2(be))

    h1 = gin_layer(h0, D0, W1_0, conv0_b1, conv0_W2, conv0_b2, ln0_g, ln0_b)
    h2 = gin_layer(h1, D1, conv1_W1, conv1_b1, conv1_W2, conv1_b2, ln1_g, ln1_b)

    out = pl.pallas_call(
        _pool_body,
        grid=(3, NB),
        in_specs=[
            pl.BlockSpec((RB, H), lambda p, i: (i, 0)),
            pl.BlockSpec((RB, 1), lambda p, i: (i, 0)),
            pl.BlockSpec((H, H), lambda p, i: (0, 0)),
            pl.BlockSpec((1, H), lambda p, i: (0, 0)),
            pl.BlockSpec((H, 1), lambda p, i: (0, 0)),
            pl.BlockSpec((1, 1), lambda p, i: (0, 0)),
            pl.BlockSpec((H, H), lambda p, i: (0, 0)),
            pl.BlockSpec((1, H), lambda p, i: (0, 0)),
            pl.BlockSpec((H, 2), lambda p, i: (0, 0)),
            pl.BlockSpec((1, 2), lambda p, i: (0, 0)),
        ],
        out_specs=pl.BlockSpec((NG, 2), lambda p, i: (0, 0)),
        out_shape=jax.ShapeDtypeStruct((NG, 2), _f32),
        scratch_shapes=[
            pltpu.VMEM((NP, 1), _f32),
            pltpu.VMEM((1, NG), _f32),
            pltpu.VMEM((1, NG), _f32),
            pltpu.VMEM((NG, H), _f32),
        ],
    )(h2, bc, gate_W1, row2(gate_b1), gate_W2, row2(gate_b2),
      cls_W1, row2(cls_b1), cls_W2, row2(cls_b2))
    return out


# speedup vs baseline: 1.8980x; 1.0377x over previous
import functools

import jax
import jax.numpy as jnp
from jax import lax
from jax.experimental import pallas as pl
from jax.experimental.pallas import tpu as pltpu
from jax.experimental.pallas import tpu_sc as plsc

N = 10000
NP = 10240
E = 320000
NG = 64
NT = 512
ED = 16
H = 128
D0 = 144
D1 = 128
RB = 512
NB = NP // RB
CH = 128
NTILES = 32
KCH = 79
EP = NTILES * KCH * CH
RPT = NP // 16

_f32 = jnp.float32


def _prep_body(x_ref, emb_ref, out_ref):
    xb = x_ref[...]
    nt = xb[:, 0:1].astype(jnp.int32)
    onehot = (nt == lax.broadcasted_iota(jnp.int32, (RB, NT), 1)).astype(_f32)
    emb_rows = jnp.dot(onehot, emb_ref[...], preferred_element_type=_f32)
    zero_col = jnp.zeros((RB, 1), _f32)
    out_ref[...] = jnp.concatenate([xb[:, 1:], emb_rows, zero_col], axis=-1)


def _mlp_body(h_ref, a0_ref, a1_ref, W1_ref, b1_ref, W2_ref, b2_ref,
              g_ref, be_ref, out_ref):
    z = h_ref[...] + a0_ref[...] + a1_ref[...]
    a = jnp.maximum(jnp.dot(z, W1_ref[...], preferred_element_type=_f32)
                    + b1_ref[...], 0.0)
    o = jnp.dot(a, W2_ref[...], preferred_element_type=_f32) + b2_ref[...]
    mu = jnp.mean(o, axis=-1, keepdims=True)
    var = jnp.mean((o - mu) ** 2, axis=-1, keepdims=True)
    o = (o - mu) * lax.rsqrt(var + 1e-5) * g_ref[...] + be_ref[...]
    out_ref[...] = jnp.maximum(o, 0.0)


def _pool_body(h_ref, bc_ref, gW1_ref, gb1_ref, gW2_ref, gb2_ref,
               cW1_ref, cb1_ref, cW2_ref, cb2_ref, out_ref,
               gate_s, gmax_s, den_s, pool_s):
    p = pl.program_id(0)
    i = pl.program_id(1)
    hb = h_ref[...]
    bc = bc_ref[...]
    seg_ids = lax.broadcasted_iota(jnp.int32, (RB, NG), 1).astype(_f32)
    seg = (bc == seg_ids).astype(_f32)

    @pl.when(p == 0)
    def _phase_gate_max():
        g = jnp.dot(jnp.maximum(jnp.dot(hb, gW1_ref[...],
                                        preferred_element_type=_f32)
                                + gb1_ref[...], 0.0),
                    gW2_ref[...], preferred_element_type=_f32) + gb2_ref[...]
        gate_s[pl.ds(i * RB, RB), :] = g
        m = jnp.max(jnp.where(seg > 0.0, g, -1e30), axis=0, keepdims=True)
        prev = jnp.where(i == 0, jnp.full((1, NG), -1e30, _f32), gmax_s[...])
        gmax_s[...] = jnp.maximum(prev, m)

    @pl.when(p == 1)
    def _phase_denom():
        g = gate_s[pl.ds(i * RB, RB), :]
        gmaxn = jnp.sum(seg * gmax_s[...], axis=1, keepdims=True)
        alpha = jnp.exp(g - gmaxn) * seg.max(axis=1, keepdims=True)
        prev = jnp.where(i == 0, jnp.zeros((1, NG), _f32), den_s[...])
        den_s[...] = prev + jnp.sum(seg * alpha, axis=0, keepdims=True)

    @pl.when(p == 2)
    def _phase_weighted_sum():
        g = gate_s[pl.ds(i * RB, RB), :]
        gmaxn = jnp.sum(seg * gmax_s[...], axis=1, keepdims=True)
        alpha = jnp.exp(g - gmaxn)
        denn = jnp.sum(seg * den_s[...], axis=1, keepdims=True)
        w = alpha / (denn + 1e-16)
        contrib = lax.dot_general(seg, w * hb, (((0,), (0,)), ((), ())),
                                  preferred_element_type=_f32)
        prev = jnp.where(i == 0, jnp.zeros((NG, H), _f32), pool_s[...])
        pool_s[...] = prev + contrib

    @pl.when((p == 2) & (i == NB - 1))
    def _classify():
        pooled = pool_s[...]
        zc = jnp.maximum(jnp.dot(pooled, cW1_ref[...],
                                 preferred_element_type=_f32) + cb1_ref[...], 0.0)
        out_ref[...] = jnp.dot(zc, cW2_ref[...],
                               preferred_element_type=_f32) + cb2_ref[...]


def _make_edge_agg(D):
    mesh = plsc.VectorSubcoreMesh(core_axis_name="c", subcore_axis_name="s")

    @functools.partial(
        pl.kernel,
        out_type=jax.ShapeDtypeStruct((2 * NP, D), _f32),
        mesh=mesh,
        compiler_params=pltpu.CompilerParams(use_tc_tiling_on_sc=False),
        scratch_types=[
            pltpu.VMEM((CH,), jnp.int32),
            pltpu.VMEM((CH,), jnp.int32),
            pltpu.VMEM((CH,), jnp.int32),
            pltpu.VMEM((CH,), jnp.int32),
            pltpu.VMEM((CH, D), _f32),
            pltpu.VMEM((CH, D), _f32),
            pltpu.VMEM_SHARED((NP, D), _f32),
            pltpu.SemaphoreType.DMA,
            pltpu.SemaphoreType.DMA,
            pltpu.SemaphoreType.DMA,
            pltpu.SemaphoreType.DMA,
        ],
    )
    def edge_agg(h_hbm, src_hbm, dst_hbm, zeros_hbm, out_hbm,
                 src_v0, src_v1, dst_v0, dst_v1, rows0, rows1, agg_sh,
                 sem0, sem1, ssem0, ssem1):
        c = lax.axis_index("c")
        s = lax.axis_index("s")
        wid = s * 2 + c
        rbase = s * RPT
        ebase = wid * (KCH * CH)

        pltpu.sync_copy(zeros_hbm, rows0)
        def zero_body(m, carry):
            pltpu.sync_copy(rows0, agg_sh.at[pl.ds(rbase + m * CH, CH)])
            return carry
        lax.fori_loop(0, RPT // CH, zero_body, 0)

        pltpu.sync_copy(src_hbm.at[pl.ds(ebase, CH)], src_v0)
        pltpu.async_copy(h_hbm.at[src_v0], rows0, sem0)
        plsc.subcore_barrier()

        def edge_body(i, carry):
            j1 = 2 * i + 1
            j2 = j1 + 1
            pltpu.sync_copy(src_hbm.at[pl.ds(ebase + j1 * CH, CH)], src_v1)
            pltpu.async_copy(h_hbm.at[src_v1], rows1, sem1)
            pltpu.sync_copy(dst_hbm.at[pl.ds(ebase + (j1 - 1) * CH, CH)], dst_v0)
            pltpu.make_async_copy(h_hbm.at[src_v0], rows0, sem0).wait()
            pltpu.async_copy(rows0, agg_sh.at[dst_v0], ssem0, add=True)
            pltpu.sync_copy(dst_hbm.at[pl.ds(ebase + j1 * CH, CH)], dst_v1)
            pltpu.make_async_copy(h_hbm.at[src_v1], rows1, sem1).wait()
            pltpu.async_copy(rows1, agg_sh.at[dst_v1], ssem1, add=True)
            pltpu.make_async_copy(rows0, agg_sh.at[dst_v0], ssem0).wait()
            @pl.when(j2 < KCH)
            def _():
                pltpu.sync_copy(src_hbm.at[pl.ds(ebase + j2 * CH, CH)], src_v0)
                pltpu.async_copy(h_hbm.at[src_v0], rows0, sem0)
            pltpu.make_async_copy(rows1, agg_sh.at[dst_v1], ssem1).wait()
            return carry
        lax.fori_loop(0, KCH // 2, edge_body, 0)

        jt = KCH - 1
        pltpu.sync_copy(dst_hbm.at[pl.ds(ebase + jt * CH, CH)], dst_v0)
        pltpu.make_async_copy(h_hbm.at[src_v0], rows0, sem0).wait()
        pltpu.sync_copy(rows0, agg_sh.at[dst_v0], add=True)
        plsc.subcore_barrier()

        pltpu.sync_copy(agg_sh.at[pl.ds(rbase, RPT)],
                        out_hbm.at[pl.ds(c * NP + rbase, RPT)])

    return edge_agg


def _row_spec(D):
    return pl.BlockSpec((RB, D), lambda i: (i, 0))


def _full(shape):
    return pl.BlockSpec(shape, lambda i: (0, 0))


def kernel(x, edge_index, batch, emb,
           conv0_W1, conv0_b1, conv0_W2, conv0_b2,
           conv1_W1, conv1_b1, conv1_W2, conv1_b2,
           ln0_g, ln0_b, ln1_g, ln1_b,
           gate_W1, gate_b1, gate_W2, gate_b2,
           cls_W1, cls_b1, cls_W2, cls_b2):
    x_p = jnp.pad(x, ((0, NP - N), (0, 0)))
    emb_p = jnp.pad(emb, ((0, NT - emb.shape[0]), (0, 0)))
    pt = KCH * CH - E // NTILES
    src_p = jnp.pad(edge_index[0].reshape(NTILES, E // NTILES),
                    ((0, 0), (0, pt))).reshape(-1)
    dst_p = jnp.pad(edge_index[1].reshape(NTILES, E // NTILES),
                    ((0, 0), (0, pt)), constant_values=N).reshape(-1)
    bc = jnp.pad(batch.astype(_f32), (0, NP - N),
                 constant_values=float(NG))[:, None]
    W1_0 = jnp.pad(conv0_W1, ((0, D0 - conv0_W1.shape[0]), (0, 0)))
    z0 = jnp.zeros((CH, D0), _f32)
    z1 = jnp.zeros((CH, D1), _f32)
    row2 = lambda v: v[None, :]

    h0 = pl.pallas_call(
        _prep_body,
        grid=(NB,),
        in_specs=[_row_spec(128), _full((NT, ED))],
        out_specs=_row_spec(D0),
        out_shape=jax.ShapeDtypeStruct((NP, D0), _f32),
    )(x_p, emb_p)

    def gin_layer(h, D, W1, b1, W2, b2, g, be):
        parts = _make_edge_agg(D)(h, src_p, dst_p, z0 if D == D0 else z1)
        return pl.pallas_call(
            _mlp_body,
            grid=(NB,),
            in_specs=[
                _row_spec(D),
                pl.BlockSpec((RB, D), lambda i: (i, 0)),
                pl.BlockSpec((RB, D), lambda i: (i + NB, 0)),
                _full((D, H)), _full((1, H)), _full((H, H)), _full((1, H)),
                _full((1, H)), _full((1, H)),
            ],
            out_specs=_row_spec(H),
            out_shape=jax.ShapeDtypeStruct((NP, H), _f32),
        )(h, parts, parts, W1, row2(b1), W2, row2(b2), row2(g), row2(be))

    h1 = gin_layer(h0, D0, W1_0, conv0_b1, conv0_W2, conv0_b2, ln0_g, ln0_b)
    h2 = gin_layer(h1, D1, conv1_W1, conv1_b1, conv1_W2, conv1_b2, ln1_g, ln1_b)

    out = pl.pallas_call(
        _pool_body,
        grid=(3, NB),
        in_specs=[
            pl.BlockSpec((RB, H), lambda p, i: (i, 0)),
            pl.BlockSpec((RB, 1), lambda p, i: (i, 0)),
            pl.BlockSpec((H, H), lambda p, i: (0, 0)),
            pl.BlockSpec((1, H), lambda p, i: (0, 0)),
            pl.BlockSpec((H, 1), lambda p, i: (0, 0)),
            pl.BlockSpec((1, 1), lambda p, i: (0, 0)),
            pl.BlockSpec((H, H), lambda p, i: (0, 0)),
            pl.BlockSpec((1, H), lambda p, i: (0, 0)),
            pl.BlockSpec((H, 2), lambda p, i: (0, 0)),
            pl.BlockSpec((1, 2), lambda p, i: (0, 0)),
        ],
        out_specs=pl.BlockSpec((NG, 2), lambda p, i: (0, 0)),
        out_shape=jax.ShapeDtypeStruct((NG, 2), _f32),
        scratch_shapes=[
            pltpu.VMEM((NP, 1), _f32),
            pltpu.VMEM((1, NG), _f32),
            pltpu.VMEM((1, NG), _f32),
            pltpu.VMEM((NG, H), _f32),
        ],
    )(h2, bc, gate_W1, row2(gate_b1), gate_W2, row2(gate_b2),
      cls_W1, row2(cls_b1), cls_W2, row2(cls_b2))
    return out
